# baseline (speedup 1.0000x reference)
import functools

import jax
import jax.numpy as jnp
from jax import lax
from jax.experimental import pallas as pl
from jax.experimental.pallas import tpu as pltpu
from jax.experimental.pallas import tpu_sc as plsc

NUM_USERS = 1000000
NUM_JOKES = 100
EMBED_DIM = 16
BATCH = 16384

NC = 2
NS = 16
NW = NC * NS
B_PER_W = BATCH // NW
CHUNK = 128
N_CHUNKS = B_PER_W // CHUNK


def _sc_gather_body(uidx_hbm, jidx_hbm, utab_hbm, jtab_hbm,
                    uout_hbm, jout_hbm,
                    uidx_v, jidx_v, urows_v, jrows_v, usem, jsem):
  wid = lax.axis_index("s") * NC + lax.axis_index("c")
  base = wid * B_PER_W
  pltpu.sync_copy(uidx_hbm.at[pl.ds(base, B_PER_W)], uidx_v)
  pltpu.sync_copy(jidx_hbm.at[pl.ds(base, B_PER_W)], jidx_v)
  copies = []
  for j in range(N_CHUNKS):
    sl = pl.ds(j * CHUNK, CHUNK)
    copies.append(pltpu.async_copy(
        utab_hbm.at[uidx_v.at[sl]], urows_v.at[sl], usem))
    copies.append(pltpu.async_copy(
        jtab_hbm.at[jidx_v.at[sl]], jrows_v.at[sl], jsem))
  for c in copies:
    c.wait()
  pltpu.sync_copy(urows_v, uout_hbm.at[pl.ds(base, B_PER_W)])
  pltpu.sync_copy(jrows_v, jout_hbm.at[pl.ds(base, B_PER_W)])


_sc_gather = functools.partial(
    pl.kernel,
    out_type=(
        jax.ShapeDtypeStruct((BATCH, EMBED_DIM), jnp.float32),
        jax.ShapeDtypeStruct((BATCH, EMBED_DIM), jnp.float32),
    ),
    mesh=plsc.VectorSubcoreMesh(
        core_axis_name="c", subcore_axis_name="s",
        num_cores=NC, num_subcores=NS),
    compiler_params=pltpu.CompilerParams(use_tc_tiling_on_sc=False),
    scratch_types=[
        pltpu.VMEM((B_PER_W,), jnp.int32),
        pltpu.VMEM((B_PER_W,), jnp.int32),
        pltpu.VMEM((B_PER_W, EMBED_DIM), jnp.float32),
        pltpu.VMEM((B_PER_W, EMBED_DIM), jnp.float32),
        pltpu.SemaphoreType.DMA,
        pltpu.SemaphoreType.DMA,
    ],
)(_sc_gather_body)


def _mlp_body(u_ref, j_ref, w1u_ref, w1j_ref, b1_ref, w2_ref, b2_ref,
              w3_ref, b3_ref, o_ref):
  dot = functools.partial(jnp.dot, preferred_element_type=jnp.float32)
  h1 = dot(u_ref[...], w1u_ref[...]) + dot(j_ref[...], w1j_ref[...])
  h1 = jnp.maximum(h1 + b1_ref[...], 0.0)
  h2 = jnp.maximum(dot(h1, w2_ref[...]) + b2_ref[...], 0.0)
  y = dot(h2, w3_ref[...]) + b3_ref[...]
  o_ref[...] = jnp.tanh(y) * 10.0


def _mlp(u_emb, j_emb, W1u, W1j, b1, W2, b2, W3, b3):
  blk = 2048
  grid = (BATCH // blk,)
  rep = lambda i: (0, 0)
  return pl.pallas_call(
      _mlp_body,
      grid=grid,
      in_specs=[
          pl.BlockSpec((blk, EMBED_DIM), lambda i: (i, 0)),
          pl.BlockSpec((blk, EMBED_DIM), lambda i: (i, 0)),
          pl.BlockSpec((EMBED_DIM, 128), rep),
          pl.BlockSpec((EMBED_DIM, 128), rep),
          pl.BlockSpec((1, 128), rep),
          pl.BlockSpec((128, 64), rep),
          pl.BlockSpec((1, 64), rep),
          pl.BlockSpec((64, 1), rep),
          pl.BlockSpec((1, 1), rep),
      ],
      out_specs=pl.BlockSpec((blk, 1), lambda i: (i, 0)),
      out_shape=jax.ShapeDtypeStruct((BATCH, 1), jnp.float32),
  )(u_emb, j_emb, W1u, W1j, b1, W2, b2, W3, b3)


def kernel(user, joke, user_table, joke_table, W1, b1, W2, b2, W3, b3):
  user = user.astype(jnp.int32)
  joke = joke.astype(jnp.int32)
  u_emb, j_emb = _sc_gather(user, joke, user_table, joke_table)
  W1u = W1[:EMBED_DIM]
  W1j = W1[EMBED_DIM:]
  return _mlp(u_emb, j_emb, W1u, W1j,
              b1.reshape(1, 128), W2, b2.reshape(1, 64),
              W3, b3.reshape(1, 1))
